# baseline (speedup 1.0000x reference)
import jax
import jax.numpy as jnp
from jax.experimental import pallas as pl

_MAX_VOXELS = 20000
_MAX_NUM_POINTS = 35
_ROWS_PER_STEP = 500


def _zero_fill(v_ref, c_ref, n_ref):
    v_ref[...] = jnp.zeros(v_ref.shape, v_ref.dtype)

    @pl.when(pl.program_id(0) == 0)
    def _():
        c_ref[...] = jnp.zeros(c_ref.shape, c_ref.dtype)
        n_ref[...] = jnp.zeros(n_ref.shape, n_ref.dtype)


def kernel(points):
    ndim = points.shape[1]
    grid = _MAX_VOXELS // _ROWS_PER_STEP
    voxels, coors, num_points = pl.pallas_call(
        _zero_fill,
        grid=(grid,),
        out_specs=(
            pl.BlockSpec((_ROWS_PER_STEP, _MAX_NUM_POINTS, ndim), lambda i: (i, 0, 0)),
            pl.BlockSpec((_MAX_VOXELS, 3), lambda i: (0, 0)),
            pl.BlockSpec((_MAX_VOXELS,), lambda i: (0,)),
        ),
        out_shape=(
            jax.ShapeDtypeStruct((_MAX_VOXELS, _MAX_NUM_POINTS, ndim), jnp.float32),
            jax.ShapeDtypeStruct((_MAX_VOXELS, 3), jnp.int32),
            jax.ShapeDtypeStruct((_MAX_VOXELS,), jnp.int32),
        ),
    )()
    return (voxels, coors, num_points)
